# baseline (speedup 1.0000x reference)
import functools

import jax
import jax.numpy as jnp
from jax import lax
from jax.experimental import pallas as pl
from jax.experimental.pallas import tpu as pltpu
from jax.experimental.pallas import tpu_sc as plsc

_THRESHOLD = 1.0
_LEAK = 0.0
_EPS = 1e-10
_L = 16
_NP = 1024
_NB = 64
_NJ = 256


def _dirf(scalar_i32, bit):
    shift = bit.bit_length() - 1
    b01 = (jnp.full((_L,), scalar_i32, jnp.int32) >> shift) & 1
    return 1.0 - 2.0 * b01.astype(jnp.float32)


def _cmpx(ks, vs, r, p, asc):
    ka, kb = ks[r], ks[p]
    va, vb = vs[r], vs[p]
    if asc:
        sel = ka <= kb
        ks[r] = jnp.minimum(ka, kb)
        ks[p] = jnp.maximum(ka, kb)
    else:
        sel = ka >= kb
        ks[r] = jnp.maximum(ka, kb)
        ks[p] = jnp.minimum(ka, kb)
    vs[r] = jnp.where(sel, va, vb)
    vs[p] = jnp.where(sel, vb, va)


def _init_pass(keys_v, vals_v):
    @plsc.parallel_loop(0, _NP // _L)
    def body(blk):
        base = blk * _L
        ks = [keys_v[base + r, :] for r in range(_L)]
        vs = [vals_v[base + r, :] for r in range(_L)]
        for k in (2, 4, 8):
            for j_exp in range(k.bit_length() - 2, -1, -1):
                j = 1 << j_exp
                for r in range(_L):
                    if r & j:
                        continue
                    _cmpx(ks, vs, r, r | j, not (r & k))
        d16 = _dirf(base, 16)
        ks = [kk * d16 for kk in ks]
        for j in (8, 4, 2, 1):
            for r in range(_L):
                if r & j:
                    continue
                _cmpx(ks, vs, r, r | j, True)
        for r in range(_L):
            keys_v[base + r, :] = ks[r] * d16
            vals_v[base + r, :] = vs[r]


def _group4(keys_v, vals_v, i0, j1, j2, asc):
    rows = (i0, i0 + j2, i0 + j1, i0 + j1 + j2)
    ks = [keys_v[r, :] for r in rows]
    vs = [vals_v[r, :] for r in rows]
    for a, b in ((0, 2), (1, 3), (0, 1), (2, 3)):
        _cmpx(ks, vs, a, b, asc)
    for n, r in enumerate(rows):
        keys_v[r, :] = ks[n]
        vals_v[r, :] = vs[n]


def _pair2(keys_v, vals_v, i0, j, asc):
    ks = [keys_v[i0, :], keys_v[i0 + j, :]]
    vs = [vals_v[i0, :], vals_v[i0 + j, :]]
    _cmpx(ks, vs, 0, 1, asc)
    keys_v[i0, :] = ks[0]
    keys_v[i0 + j, :] = ks[1]
    vals_v[i0, :] = vs[0]
    vals_v[i0 + j, :] = vs[1]


def _expand(q, j_exp):
    return ((q >> j_exp) << (j_exp + 1)) | (q & ((1 << j_exp) - 1))


def _mem_fused(keys_v, vals_v, k, j1):
    j2 = j1 // 2
    j1e = j1.bit_length() - 1
    j2e = j2.bit_length() - 1
    groups = k // 4
    if k == _NP:
        @plsc.parallel_loop(0, groups, unroll=2)
        def inner(g):
            x = _expand(_expand(g, j2e), j1e)
            _group4(keys_v, vals_v, x, j1, j2, True)
    else:
        def outer(bp, carry):
            base = 2 * bp * k
            @plsc.parallel_loop(0, groups, unroll=2)
            def inner(g):
                x = _expand(_expand(g, j2e), j1e)
                _group4(keys_v, vals_v, base + x, j1, j2, True)
                _group4(keys_v, vals_v, base + k + x, j1, j2, False)
            return carry
        lax.fori_loop(0, _NP // (2 * k), outer, 0)


def _group8(keys_v, vals_v, i0, j1, asc):
    j2, j3 = j1 // 2, j1 // 4
    rows = [i0 + b2 * j1 + b1 * j2 + b0 * j3
            for b2 in (0, 1) for b1 in (0, 1) for b0 in (0, 1)]
    ks = [keys_v[r, :] for r in rows]
    vs = [vals_v[r, :] for r in rows]
    for m in (0, 1, 2, 3):
        _cmpx(ks, vs, m, m | 4, asc)
    for m in (0, 1, 4, 5):
        _cmpx(ks, vs, m, m | 2, asc)
    for m in (0, 2, 4, 6):
        _cmpx(ks, vs, m, m | 1, asc)
    for n, r in enumerate(rows):
        keys_v[r, :] = ks[n]
        vals_v[r, :] = vs[n]


def _mem_fused3(keys_v, vals_v, k, j1):
    j1e = j1.bit_length() - 1
    groups = k // 8
    if k == _NP:
        @plsc.parallel_loop(0, groups, unroll=2)
        def inner(g):
            x = _expand(_expand(_expand(g, j1e - 2), j1e - 1), j1e)
            _group8(keys_v, vals_v, x, j1, True)
    else:
        def outer(bp, carry):
            base = 2 * bp * k
            @plsc.parallel_loop(0, groups)
            def inner(g):
                x = _expand(_expand(_expand(g, j1e - 2), j1e - 1), j1e)
                _group8(keys_v, vals_v, base + x, j1, True)
                _group8(keys_v, vals_v, base + k + x, j1, False)
            return carry
        lax.fori_loop(0, _NP // (2 * k), outer, 0)


def _mem_single(keys_v, vals_v, k, j):
    je = j.bit_length() - 1
    pairs = k // 2
    if k == _NP:
        @plsc.parallel_loop(0, pairs, unroll=4)
        def inner(g):
            x = _expand(g, je)
            _pair2(keys_v, vals_v, x, j, True)
    else:
        def outer(bp, carry):
            base = 2 * bp * k
            @plsc.parallel_loop(0, pairs, unroll=2)
            def inner(g):
                x = _expand(g, je)
                _pair2(keys_v, vals_v, base + x, j, True)
                _pair2(keys_v, vals_v, base + k + x, j, False)
            return carry
        lax.fori_loop(0, _NP // (2 * k), outer, 0)


def _reg_block(keys_v, vals_v, base, asc):
    ks = [keys_v[base + r, :] for r in range(_L)]
    vs = [vals_v[base + r, :] for r in range(_L)]
    for j in (8, 4, 2, 1):
        for r in range(_L):
            if r & j:
                continue
            _cmpx(ks, vs, r, r | j, asc)
    for r in range(_L):
        keys_v[base + r, :] = ks[r]
        vals_v[base + r, :] = vs[r]


def _reg_pass(keys_v, vals_v, k):
    if k == _NP:
        @plsc.parallel_loop(0, _NP // _L)
        def body(blk):
            _reg_block(keys_v, vals_v, blk * _L, True)
    else:
        run = k // _L
        def outer(rp, carry):
            b0 = rp * 2 * run
            @plsc.parallel_loop(0, run)
            def inner_a(q):
                _reg_block(keys_v, vals_v, (b0 + q) * _L, True)
            @plsc.parallel_loop(0, run)
            def inner_d(q):
                _reg_block(keys_v, vals_v, (b0 + run + q) * _L, False)
            return carry
        lax.fori_loop(0, _NP // _L // (2 * run), outer, 0)


def _final_pass(keys_v, vals_v):
    inf = jnp.full((_L,), jnp.inf, jnp.float32)
    zero = jnp.zeros((_L,), jnp.float32)
    one = jnp.ones((_L,), jnp.float32)
    unroll = 4

    def step(i, c):
        cw, cwt, num_p, den_p, run_n, run_d = c
        s = keys_v[i, :]
        w = vals_v[i, :]
        bet = num_p * run_d < run_n * den_p
        rn = jnp.where(bet, num_p, run_n)
        rd = jnp.where(bet, den_p, run_d)
        ok_u = num_p <= s * den_p
        run_n = jnp.where(ok_u, rn, run_n)
        run_d = jnp.where(ok_u, rd, run_d)
        cw = cw + w
        cwt = cwt + w * s
        num = (_THRESHOLD - _LEAK) + cwt
        num = jnp.where(num >= s * cw, num, inf)
        num = jnp.where(cw > _EPS, num, inf)
        return (cw, cwt, num, cw, run_n, run_d)

    @plsc.parallel_loop(0, _NP // unroll,
                        carry=(zero, zero, inf, one, inf, one))
    def body(i4, c):
        for r in range(unroll):
            c = step(i4 * unroll + r, c)
        return c

    cw, cwt, num_p, den_p, run_n, run_d = body
    bet = num_p * run_d < run_n * den_p
    run_n = jnp.where(bet, num_p, run_n)
    run_d = jnp.where(bet, den_p, run_d)
    return run_n / run_d


def _sc_body(spikes_hbm, weights_hbm, out_hbm, keys_v, vals_v, out_v):
    wid = lax.axis_index("s") * 2 + lax.axis_index("c")
    jt = wid % (_NJ // _L)
    bg = wid // (_NJ // _L)
    col0 = jt * _L

    def tile(step, carry):
        b = bg * (_NB // 2) + step
        pltpu.sync_copy(spikes_hbm.at[b, :, pl.ds(col0, _L)], keys_v)
        pltpu.sync_copy(weights_hbm.at[:, pl.ds(col0, _L)], vals_v)
        _init_pass(keys_v, vals_v)
        for k_exp in range(5, _NP.bit_length()):
            k = 1 << k_exp
            strides = [1 << e for e in range(k_exp - 1, 3, -1)]
            while strides:
                if len(strides) in (4, 2):
                    _mem_fused(keys_v, vals_v, k, strides[0])
                    strides = strides[2:]
                elif len(strides) >= 3:
                    _mem_fused3(keys_v, vals_v, k, strides[0])
                    strides = strides[3:]
                else:
                    _mem_single(keys_v, vals_v, k, strides[0])
                    strides = strides[1:]
            _reg_pass(keys_v, vals_v, k)
        out_v[...] = _final_pass(keys_v, vals_v)
        pltpu.sync_copy(out_v, out_hbm.at[b, pl.ds(col0, _L)])
        return carry

    lax.fori_loop(0, _NB // 2, tile, 0)


@jax.jit
def kernel(input_spikes, input_weights):
    f = pl.kernel(
        _sc_body,
        out_type=jax.ShapeDtypeStruct((_NB, _NJ), jnp.float32),
        mesh=plsc.VectorSubcoreMesh(core_axis_name="c", subcore_axis_name="s"),
        compiler_params=pltpu.CompilerParams(use_tc_tiling_on_sc=False),
        scratch_types=[
            pltpu.VMEM((_NP, _L), jnp.float32),
            pltpu.VMEM((_NP, _L), jnp.float32),
            pltpu.VMEM((_L,), jnp.float32),
        ],
    )
    return f(input_spikes, input_weights)


# speedup vs baseline: 1.0907x; 1.0907x over previous
import functools

import jax
import jax.numpy as jnp
from jax import lax
from jax.experimental import pallas as pl
from jax.experimental.pallas import tpu as pltpu
from jax.experimental.pallas import tpu_sc as plsc

_THRESHOLD = 1.0
_LEAK = 0.0
_EPS = 1e-10
_L = 16
_NP = 1024
_NB = 64
_NJ = 256


def _dirf(scalar_i32, bit):
    shift = bit.bit_length() - 1
    b01 = (jnp.full((_L,), scalar_i32, jnp.int32) >> shift) & 1
    return 1.0 - 2.0 * b01.astype(jnp.float32)


def _cmpx(ks, vs, r, p, asc):
    ka, kb = ks[r], ks[p]
    va, vb = vs[r], vs[p]
    if asc:
        sel = ka <= kb
        ks[r] = jnp.minimum(ka, kb)
        ks[p] = jnp.maximum(ka, kb)
    else:
        sel = ka >= kb
        ks[r] = jnp.maximum(ka, kb)
        ks[p] = jnp.minimum(ka, kb)
    vs[r] = jnp.where(sel, va, vb)
    vs[p] = jnp.where(sel, vb, va)


def _init_pass(keys_v, vals_v):
    @plsc.parallel_loop(0, _NP // _L)
    def body(blk):
        base = blk * _L
        ks = [keys_v[base + r, :] for r in range(_L)]
        vs = [vals_v[base + r, :] for r in range(_L)]
        for k in (2, 4, 8):
            for j_exp in range(k.bit_length() - 2, -1, -1):
                j = 1 << j_exp
                for r in range(_L):
                    if r & j:
                        continue
                    _cmpx(ks, vs, r, r | j, not (r & k))
        d16 = _dirf(base, 16)
        ks = [kk * d16 for kk in ks]
        for j in (8, 4, 2, 1):
            for r in range(_L):
                if r & j:
                    continue
                _cmpx(ks, vs, r, r | j, True)
        for r in range(_L):
            keys_v[base + r, :] = ks[r] * d16
            vals_v[base + r, :] = vs[r]


def _group4(keys_v, vals_v, i0, j1, j2, asc):
    rows = (i0, i0 + j2, i0 + j1, i0 + j1 + j2)
    ks = [keys_v[r, :] for r in rows]
    vs = [vals_v[r, :] for r in rows]
    for a, b in ((0, 2), (1, 3), (0, 1), (2, 3)):
        _cmpx(ks, vs, a, b, asc)
    for n, r in enumerate(rows):
        keys_v[r, :] = ks[n]
        vals_v[r, :] = vs[n]


def _pair2(keys_v, vals_v, i0, j, asc):
    ks = [keys_v[i0, :], keys_v[i0 + j, :]]
    vs = [vals_v[i0, :], vals_v[i0 + j, :]]
    _cmpx(ks, vs, 0, 1, asc)
    keys_v[i0, :] = ks[0]
    keys_v[i0 + j, :] = ks[1]
    vals_v[i0, :] = vs[0]
    vals_v[i0 + j, :] = vs[1]


def _expand(q, j_exp):
    return ((q >> j_exp) << (j_exp + 1)) | (q & ((1 << j_exp) - 1))


def _mem_fused(keys_v, vals_v, k, j1):
    j2 = j1 // 2
    j1e = j1.bit_length() - 1
    j2e = j2.bit_length() - 1
    groups = k // 4
    if k == _NP:
        @plsc.parallel_loop(0, groups, unroll=2)
        def inner(g):
            x = _expand(_expand(g, j2e), j1e)
            _group4(keys_v, vals_v, x, j1, j2, True)
    else:
        def outer(bp, carry):
            base = 2 * bp * k
            @plsc.parallel_loop(0, groups, unroll=2)
            def inner(g):
                x = _expand(_expand(g, j2e), j1e)
                _group4(keys_v, vals_v, base + x, j1, j2, True)
                _group4(keys_v, vals_v, base + k + x, j1, j2, False)
            return carry
        lax.fori_loop(0, _NP // (2 * k), outer, 0)


def _group8(keys_v, vals_v, i0, j1, asc):
    j2, j3 = j1 // 2, j1 // 4
    rows = [i0 + b2 * j1 + b1 * j2 + b0 * j3
            for b2 in (0, 1) for b1 in (0, 1) for b0 in (0, 1)]
    ks = [keys_v[r, :] for r in rows]
    vs = [vals_v[r, :] for r in rows]
    for m in (0, 1, 2, 3):
        _cmpx(ks, vs, m, m | 4, asc)
    for m in (0, 1, 4, 5):
        _cmpx(ks, vs, m, m | 2, asc)
    for m in (0, 2, 4, 6):
        _cmpx(ks, vs, m, m | 1, asc)
    for n, r in enumerate(rows):
        keys_v[r, :] = ks[n]
        vals_v[r, :] = vs[n]


def _mem_fused3(keys_v, vals_v, k, j1):
    j1e = j1.bit_length() - 1
    groups = k // 8
    if k == _NP:
        @plsc.parallel_loop(0, groups, unroll=2)
        def inner(g):
            x = _expand(_expand(_expand(g, j1e - 2), j1e - 1), j1e)
            _group8(keys_v, vals_v, x, j1, True)
    else:
        def outer(bp, carry):
            base = 2 * bp * k
            @plsc.parallel_loop(0, groups)
            def inner(g):
                x = _expand(_expand(_expand(g, j1e - 2), j1e - 1), j1e)
                _group8(keys_v, vals_v, base + x, j1, True)
                _group8(keys_v, vals_v, base + k + x, j1, False)
            return carry
        lax.fori_loop(0, _NP // (2 * k), outer, 0)


def _mem_single(keys_v, vals_v, k, j):
    je = j.bit_length() - 1
    pairs = k // 2
    if k == _NP:
        @plsc.parallel_loop(0, pairs, unroll=4)
        def inner(g):
            x = _expand(g, je)
            _pair2(keys_v, vals_v, x, j, True)
    else:
        def outer(bp, carry):
            base = 2 * bp * k
            @plsc.parallel_loop(0, pairs, unroll=2)
            def inner(g):
                x = _expand(g, je)
                _pair2(keys_v, vals_v, base + x, j, True)
                _pair2(keys_v, vals_v, base + k + x, j, False)
            return carry
        lax.fori_loop(0, _NP // (2 * k), outer, 0)


def _reg_pass(keys_v, vals_v, k):
    fold = k != _NP
    @plsc.parallel_loop(0, _NP // _L)
    def body(blk):
        base = blk * _L
        ks = [keys_v[base + r, :] for r in range(_L)]
        vs = [vals_v[base + r, :] for r in range(_L)]
        if fold:
            d = _dirf(base, k)
            ks = [kk * d for kk in ks]
        for j in (8, 4, 2, 1):
            for r in range(_L):
                if r & j:
                    continue
                _cmpx(ks, vs, r, r | j, True)
        for r in range(_L):
            keys_v[base + r, :] = ks[r] * d if fold else ks[r]
            vals_v[base + r, :] = vs[r]


def _final_pass(keys_v, vals_v):
    inf = jnp.full((_L,), jnp.inf, jnp.float32)
    zero = jnp.zeros((_L,), jnp.float32)
    one = jnp.ones((_L,), jnp.float32)
    unroll = 4

    def step(i, c):
        cw, cwt, num_p, den_p, run_n, run_d = c
        s = keys_v[i, :]
        w = vals_v[i, :]
        bet = num_p * run_d < run_n * den_p
        rn = jnp.where(bet, num_p, run_n)
        rd = jnp.where(bet, den_p, run_d)
        ok_u = num_p <= s * den_p
        run_n = jnp.where(ok_u, rn, run_n)
        run_d = jnp.where(ok_u, rd, run_d)
        cw = cw + w
        cwt = cwt + w * s
        num = (_THRESHOLD - _LEAK) + cwt
        num = jnp.where(num >= s * cw, num, inf)
        num = jnp.where(cw > _EPS, num, inf)
        return (cw, cwt, num, cw, run_n, run_d)

    def body(i4, c):
        for r in range(unroll):
            c = step(i4 * unroll + r, c)
        return c

    cw, cwt, num_p, den_p, run_n, run_d = lax.fori_loop(
        0, _NP // unroll, body, (zero, zero, inf, one, inf, one))
    bet = num_p * run_d < run_n * den_p
    run_n = jnp.where(bet, num_p, run_n)
    run_d = jnp.where(bet, den_p, run_d)
    return run_n / run_d


def _sc_body(spikes_hbm, weights_hbm, out_hbm,
             keys_a, vals_a, keys_b, vals_b, out_v, sem_a, sem_b):
    wid = lax.axis_index("s") * 2 + lax.axis_index("c")
    jt = wid % (_NJ // _L)
    bg = wid // (_NJ // _L)
    col0 = jt * _L

    def fire(b, kv, vv, sem):
        pltpu.async_copy(spikes_hbm.at[b, :, pl.ds(col0, _L)], kv, sem)
        pltpu.async_copy(weights_hbm.at[:, pl.ds(col0, _L)], vv, sem)

    def drain(b, kv, vv, sem):
        pltpu.make_async_copy(spikes_hbm.at[b, :, pl.ds(col0, _L)], kv, sem).wait()
        pltpu.make_async_copy(weights_hbm.at[:, pl.ds(col0, _L)], vv, sem).wait()

    def compute(b, keys_v, vals_v):
        _init_pass(keys_v, vals_v)
        for k_exp in range(5, _NP.bit_length()):
            k = 1 << k_exp
            strides = [1 << e for e in range(k_exp - 1, 3, -1)]
            while strides:
                if len(strides) in (4, 2):
                    _mem_fused(keys_v, vals_v, k, strides[0])
                    strides = strides[2:]
                elif len(strides) >= 3:
                    _mem_fused3(keys_v, vals_v, k, strides[0])
                    strides = strides[3:]
                else:
                    _mem_single(keys_v, vals_v, k, strides[0])
                    strides = strides[1:]
            _reg_pass(keys_v, vals_v, k)
        out_v[...] = _final_pass(keys_v, vals_v)
        pltpu.sync_copy(out_v, out_hbm.at[b, pl.ds(col0, _L)])

    b00 = bg * (_NB // 2)
    fire(b00, keys_a, vals_a, sem_a)

    def pair(t, carry):
        b0 = b00 + 2 * t
        fire(b0 + 1, keys_b, vals_b, sem_b)
        drain(b0, keys_a, vals_a, sem_a)
        compute(b0, keys_a, vals_a)

        @pl.when(t < _NB // 4 - 1)
        def _prefetch():
            fire(b0 + 2, keys_a, vals_a, sem_a)

        drain(b0 + 1, keys_b, vals_b, sem_b)
        compute(b0 + 1, keys_b, vals_b)
        return carry

    lax.fori_loop(0, _NB // 4, pair, 0)


@jax.jit
def kernel(input_spikes, input_weights):
    f = pl.kernel(
        _sc_body,
        out_type=jax.ShapeDtypeStruct((_NB, _NJ), jnp.float32),
        mesh=plsc.VectorSubcoreMesh(core_axis_name="c", subcore_axis_name="s"),
        compiler_params=pltpu.CompilerParams(use_tc_tiling_on_sc=False),
        scratch_types=[
            pltpu.VMEM((_NP, _L), jnp.float32),
            pltpu.VMEM((_NP, _L), jnp.float32),
            pltpu.VMEM((_NP, _L), jnp.float32),
            pltpu.VMEM((_NP, _L), jnp.float32),
            pltpu.VMEM((_L,), jnp.float32),
            pltpu.SemaphoreType.DMA,
            pltpu.SemaphoreType.DMA,
        ],
    )
    return f(input_spikes, input_weights)


# speedup vs baseline: 1.1166x; 1.0237x over previous
import jax
import jax.numpy as jnp
from jax import lax
from jax.experimental import pallas as pl
from jax.experimental.pallas import tpu as pltpu
from jax.experimental.pallas import tpu_sc as plsc

_THRESHOLD = 1.0
_LEAK = 0.0
_EPS = 1e-10
_L = 16
_NP = 1024
_NB = 64
_NJ = 256


def _dirf(scalar_i32, bit):
    shift = bit.bit_length() - 1
    b01 = (jnp.full((_L,), scalar_i32, jnp.int32) >> shift) & 1
    return 1.0 - 2.0 * b01.astype(jnp.float32)


def _cmpx(ks, vs, r, p, asc):
    ka, kb = ks[r], ks[p]
    va, vb = vs[r], vs[p]
    if asc:
        sel = ka <= kb
        ks[r] = jnp.minimum(ka, kb)
        ks[p] = jnp.maximum(ka, kb)
    else:
        sel = ka >= kb
        ks[r] = jnp.maximum(ka, kb)
        ks[p] = jnp.minimum(ka, kb)
    vs[r] = jnp.where(sel, va, vb)
    vs[p] = jnp.where(sel, vb, va)


def _oddeven_merge(lo, hi, r):
    step = r * 2
    if step < hi - lo:
        yield from _oddeven_merge(lo, hi, step)
        yield from _oddeven_merge(lo + r, hi, step)
        for i in range(lo + r, hi - r, step):
            yield (i, i + r)
    else:
        yield (lo, lo + r)


def _oems(lo, hi):
    if (hi - lo) >= 1:
        mid = lo + ((hi - lo) // 2)
        yield from _oems(lo, mid)
        yield from _oems(mid + 1, hi)
        yield from _oddeven_merge(lo, hi, 1)


_BATCHER16 = list(_oems(0, _L - 1))


def _init_pass(keys_v, vals_v):
    @plsc.parallel_loop(0, _NP // _L)
    def body(blk):
        base = blk * _L
        d16 = _dirf(base, 16)
        ks = [keys_v[base + r, :] * d16 for r in range(_L)]
        vs = [vals_v[base + r, :] for r in range(_L)]
        for a, b in _BATCHER16:
            _cmpx(ks, vs, a, b, True)
        for r in range(_L):
            keys_v[base + r, :] = ks[r] * d16
            vals_v[base + r, :] = vs[r]


def _group4(keys_v, vals_v, i0, j1, j2, asc):
    rows = (i0, i0 + j2, i0 + j1, i0 + j1 + j2)
    ks = [keys_v[r, :] for r in rows]
    vs = [vals_v[r, :] for r in rows]
    for a, b in ((0, 2), (1, 3), (0, 1), (2, 3)):
        _cmpx(ks, vs, a, b, asc)
    for n, r in enumerate(rows):
        keys_v[r, :] = ks[n]
        vals_v[r, :] = vs[n]


def _pair2(keys_v, vals_v, i0, j, asc):
    ks = [keys_v[i0, :], keys_v[i0 + j, :]]
    vs = [vals_v[i0, :], vals_v[i0 + j, :]]
    _cmpx(ks, vs, 0, 1, asc)
    keys_v[i0, :] = ks[0]
    keys_v[i0 + j, :] = ks[1]
    vals_v[i0, :] = vs[0]
    vals_v[i0 + j, :] = vs[1]


def _expand(q, j_exp):
    return ((q >> j_exp) << (j_exp + 1)) | (q & ((1 << j_exp) - 1))


def _mem_fused(keys_v, vals_v, k, j1):
    j2 = j1 // 2
    j1e = j1.bit_length() - 1
    j2e = j2.bit_length() - 1
    groups = k // 4
    if k == _NP:
        @plsc.parallel_loop(0, groups, unroll=2)
        def inner(g):
            x = _expand(_expand(g, j2e), j1e)
            _group4(keys_v, vals_v, x, j1, j2, True)
    else:
        def outer(bp, carry):
            base = 2 * bp * k
            @plsc.parallel_loop(0, groups, unroll=2)
            def inner(g):
                x = _expand(_expand(g, j2e), j1e)
                _group4(keys_v, vals_v, base + x, j1, j2, True)
                _group4(keys_v, vals_v, base + k + x, j1, j2, False)
            return carry
        lax.fori_loop(0, _NP // (2 * k), outer, 0)


def _group8(keys_v, vals_v, i0, j1, asc):
    j2, j3 = j1 // 2, j1 // 4
    rows = [i0 + b2 * j1 + b1 * j2 + b0 * j3
            for b2 in (0, 1) for b1 in (0, 1) for b0 in (0, 1)]
    ks = [keys_v[r, :] for r in rows]
    vs = [vals_v[r, :] for r in rows]
    for m in (0, 1, 2, 3):
        _cmpx(ks, vs, m, m | 4, asc)
    for m in (0, 1, 4, 5):
        _cmpx(ks, vs, m, m | 2, asc)
    for m in (0, 2, 4, 6):
        _cmpx(ks, vs, m, m | 1, asc)
    for n, r in enumerate(rows):
        keys_v[r, :] = ks[n]
        vals_v[r, :] = vs[n]


def _mem_fused3(keys_v, vals_v, k, j1):
    j1e = j1.bit_length() - 1
    groups = k // 8
    if k == _NP:
        @plsc.parallel_loop(0, groups, unroll=2)
        def inner(g):
            x = _expand(_expand(_expand(g, j1e - 2), j1e - 1), j1e)
            _group8(keys_v, vals_v, x, j1, True)
    else:
        def outer(bp, carry):
            base = 2 * bp * k
            @plsc.parallel_loop(0, groups)
            def inner(g):
                x = _expand(_expand(_expand(g, j1e - 2), j1e - 1), j1e)
                _group8(keys_v, vals_v, base + x, j1, True)
                _group8(keys_v, vals_v, base + k + x, j1, False)
            return carry
        lax.fori_loop(0, _NP // (2 * k), outer, 0)


def _mem_single(keys_v, vals_v, k, j):
    je = j.bit_length() - 1
    pairs = k // 2
    if k == _NP:
        @plsc.parallel_loop(0, pairs, unroll=4)
        def inner(g):
            x = _expand(g, je)
            _pair2(keys_v, vals_v, x, j, True)
    else:
        def outer(bp, carry):
            base = 2 * bp * k
            @plsc.parallel_loop(0, pairs, unroll=2)
            def inner(g):
                x = _expand(g, je)
                _pair2(keys_v, vals_v, base + x, j, True)
                _pair2(keys_v, vals_v, base + k + x, j, False)
            return carry
        lax.fori_loop(0, _NP // (2 * k), outer, 0)


def _reg_pass(keys_v, vals_v, k):
    fold = k != _NP
    @plsc.parallel_loop(0, _NP // _L)
    def body(blk):
        base = blk * _L
        ks = [keys_v[base + r, :] for r in range(_L)]
        vs = [vals_v[base + r, :] for r in range(_L)]
        if fold:
            d = _dirf(base, k)
            ks = [kk * d for kk in ks]
        for j in (8, 4, 2, 1):
            for r in range(_L):
                if r & j:
                    continue
                _cmpx(ks, vs, r, r | j, True)
        for r in range(_L):
            keys_v[base + r, :] = ks[r] * d if fold else ks[r]
            vals_v[base + r, :] = vs[r]


def _final_pass(keys_v, vals_v):
    inf = jnp.full((_L,), jnp.inf, jnp.float32)
    zero = jnp.zeros((_L,), jnp.float32)
    one = jnp.ones((_L,), jnp.float32)
    unroll = 4

    def step(i, c):
        cw, cwt, num_p, den_p, run_n, run_d = c
        s = keys_v[i, :]
        w = vals_v[i, :]
        bet = num_p * run_d < run_n * den_p
        rn = jnp.where(bet, num_p, run_n)
        rd = jnp.where(bet, den_p, run_d)
        ok_u = num_p <= s * den_p
        run_n = jnp.where(ok_u, rn, run_n)
        run_d = jnp.where(ok_u, rd, run_d)
        cw = cw + w
        cwt = cwt + w * s
        num = (_THRESHOLD - _LEAK) + cwt
        num = jnp.where(num >= s * cw, num, inf)
        num = jnp.where(cw > _EPS, num, inf)
        return (cw, cwt, num, cw, run_n, run_d)

    def body(i4, c):
        for r in range(unroll):
            c = step(i4 * unroll + r, c)
        return c

    cw, cwt, num_p, den_p, run_n, run_d = lax.fori_loop(
        0, _NP // unroll, body, (zero, zero, inf, one, inf, one))
    bet = num_p * run_d < run_n * den_p
    run_n = jnp.where(bet, num_p, run_n)
    run_d = jnp.where(bet, den_p, run_d)
    return run_n / run_d


def _sc_body(spikes_hbm, weights_hbm, out_hbm,
             keys_a, vals_a, keys_b, vals_b, out_v, sem_a, sem_b):
    wid = lax.axis_index("s") * 2 + lax.axis_index("c")
    jt = wid % (_NJ // _L)
    bg = wid // (_NJ // _L)
    col0 = jt * _L

    def fire(b, kv, vv, sem):
        pltpu.async_copy(spikes_hbm.at[b, :, pl.ds(col0, _L)], kv, sem)
        pltpu.async_copy(weights_hbm.at[:, pl.ds(col0, _L)], vv, sem)

    def drain(b, kv, vv, sem):
        pltpu.make_async_copy(spikes_hbm.at[b, :, pl.ds(col0, _L)], kv, sem).wait()
        pltpu.make_async_copy(weights_hbm.at[:, pl.ds(col0, _L)], vv, sem).wait()

    def compute(b, keys_v, vals_v):
        _init_pass(keys_v, vals_v)
        for k_exp in range(5, _NP.bit_length()):
            k = 1 << k_exp
            strides = [1 << e for e in range(k_exp - 1, 3, -1)]
            while strides:
                if len(strides) in (4, 2):
                    _mem_fused(keys_v, vals_v, k, strides[0])
                    strides = strides[2:]
                elif len(strides) >= 3:
                    _mem_fused3(keys_v, vals_v, k, strides[0])
                    strides = strides[3:]
                else:
                    _mem_single(keys_v, vals_v, k, strides[0])
                    strides = strides[1:]
            _reg_pass(keys_v, vals_v, k)
        out_v[...] = _final_pass(keys_v, vals_v)
        pltpu.sync_copy(out_v, out_hbm.at[b, pl.ds(col0, _L)])

    b00 = bg * (_NB // 2)
    fire(b00, keys_a, vals_a, sem_a)

    def pair(t, carry):
        b0 = b00 + 2 * t
        fire(b0 + 1, keys_b, vals_b, sem_b)
        drain(b0, keys_a, vals_a, sem_a)
        compute(b0, keys_a, vals_a)

        @pl.when(t < _NB // 4 - 1)
        def _prefetch():
            fire(b0 + 2, keys_a, vals_a, sem_a)

        drain(b0 + 1, keys_b, vals_b, sem_b)
        compute(b0 + 1, keys_b, vals_b)
        return carry

    lax.fori_loop(0, _NB // 4, pair, 0)


@jax.jit
def kernel(input_spikes, input_weights):
    f = pl.kernel(
        _sc_body,
        out_type=jax.ShapeDtypeStruct((_NB, _NJ), jnp.float32),
        mesh=plsc.VectorSubcoreMesh(core_axis_name="c", subcore_axis_name="s"),
        compiler_params=pltpu.CompilerParams(use_tc_tiling_on_sc=False),
        scratch_types=[
            pltpu.VMEM((_NP, _L), jnp.float32),
            pltpu.VMEM((_NP, _L), jnp.float32),
            pltpu.VMEM((_NP, _L), jnp.float32),
            pltpu.VMEM((_NP, _L), jnp.float32),
            pltpu.VMEM((_L,), jnp.float32),
            pltpu.SemaphoreType.DMA,
            pltpu.SemaphoreType.DMA,
        ],
    )
    return f(input_spikes, input_weights)
